# Initial kernel scaffold; baseline (speedup 1.0000x reference)
import functools

import jax
import jax.numpy as jnp
import numpy as np
from jax import lax
from jax.experimental import pallas as pl
from jax.experimental.pallas import tpu as pltpu
from jax.experimental.pallas import tpu_sc as plsc

N_NODE = 10000
N_EDGE = 160000
D = 256
N_HEADS = 8
D_HEAD = 32
D_OUT = 64

SC_CORES = 2
SC_TILES = 16
W_EDGE = 80
EDGES_PER_TILE = N_EDGE // SC_TILES
N_WIN = EDGES_PER_TILE // W_EDGE
ROWS_PER_TILE = N_NODE // SC_TILES
ZROWS = 125

_f32 = jnp.float32


def _proj_body(x_ref, w_ref, b_ref, a_ref, h_ref, s_ref):
    h = jnp.dot(x_ref[...], w_ref[...], preferred_element_type=_f32)
    h = h + b_ref[...]
    h_ref[...] = h
    s_ref[...] = jnp.dot(h, a_ref[...], preferred_element_type=_f32)


def _project(x, w, b, a_mat):
    blk = 1000
    grid = (N_NODE // blk,)
    return pl.pallas_call(
        _proj_body,
        grid=grid,
        in_specs=[
            pl.BlockSpec((blk, D), lambda i: (i, 0)),
            pl.BlockSpec((D, D), lambda i: (0, 0)),
            pl.BlockSpec((1, D), lambda i: (0, 0)),
            pl.BlockSpec((D, 16), lambda i: (0, 0)),
        ],
        out_specs=[
            pl.BlockSpec((blk, D), lambda i: (i, 0)),
            pl.BlockSpec((blk, 16), lambda i: (i, 0)),
        ],
        out_shape=[
            jax.ShapeDtypeStruct((N_NODE, D), _f32),
            jax.ShapeDtypeStruct((N_NODE, 16), _f32),
        ],
    )(x, w, b, a_mat)


def _sc_zero_acc(acc, accd, zbuf, zdbuf, sid):
    @pl.loop(0, 5)
    def _(r):
        base = sid * ROWS_PER_TILE + r * ZROWS
        pltpu.sync_copy(zbuf, acc.at[pl.ds(base, ZROWS)])
        pltpu.sync_copy(zdbuf, accd.at[pl.ds(base, ZROWS)])


def _sc_phase(phase, h2, s_src, s_dst, src_hbm, dst_hbm, acc, accd,
              srcbuf, dstbuf, idx2buf, asrcbuf, adstbuf, exbuf, rowsbuf,
              sid, cid):
    ex_base = phase * 8 + 4 * cid

    @pl.loop(0, N_WIN)
    def _(w):
        off = sid * EDGES_PER_TILE + w * W_EDGE
        pltpu.sync_copy(src_hbm.at[pl.ds(off, W_EDGE)], srcbuf)
        pltpu.sync_copy(dst_hbm.at[pl.ds(off, W_EDGE)], dstbuf)
        pltpu.sync_copy(s_src.at[srcbuf], asrcbuf)
        pltpu.sync_copy(s_dst.at[dstbuf], adstbuf)

        @pl.loop(0, W_EDGE, step=16)
        def _(i):
            sv = srcbuf[pl.ds(i, 16)]
            idx2buf[pl.ds(i, 16)] = sv + sv + cid

        pltpu.sync_copy(h2.at[idx2buf], rowsbuf)

        @pl.loop(0, W_EDGE)
        def _(e):
            av = asrcbuf[e, :] + adstbuf[e, :]
            av = jnp.where(av > 0.0, av, 0.2 * av)
            exbuf[e, :] = jnp.exp(av)

        @pl.loop(0, W_EDGE)
        def _(e):
            for hh in range(4):
                sc = exbuf[e, ex_base + hh]
                for k in range(2):
                    sl = pl.ds(hh * 32 + k * 16, 16)
                    rowsbuf[e, sl] = rowsbuf[e, sl] * sc

        pltpu.sync_copy(rowsbuf, acc.at[dstbuf], add=True)
        pltpu.sync_copy(exbuf, accd.at[dstbuf], add=True)


def _sc_flush(acc, accd, ou_hbm, den_hbm, sid, cid):
    base = sid * ROWS_PER_TILE
    pltpu.sync_copy(acc.at[pl.ds(base, ROWS_PER_TILE)],
                    ou_hbm.at[cid].at[pl.ds(base, ROWS_PER_TILE)])
    pltpu.sync_copy(accd.at[pl.ds(base, ROWS_PER_TILE)],
                    den_hbm.at[cid].at[pl.ds(base, ROWS_PER_TILE)])


def _sc_conv(h_au2, h_pa2, au_s, pa_s, src_w, dst_w, src_r, dst_r):
    mesh = plsc.VectorSubcoreMesh(core_axis_name="c", subcore_axis_name="s")

    @functools.partial(
        pl.kernel,
        mesh=mesh,
        out_type=[
            jax.ShapeDtypeStruct((SC_CORES, N_NODE, 128), _f32),
            jax.ShapeDtypeStruct((SC_CORES, N_NODE, 16), _f32),
            jax.ShapeDtypeStruct((SC_CORES, N_NODE, 128), _f32),
            jax.ShapeDtypeStruct((SC_CORES, N_NODE, 16), _f32),
        ],
        scratch_types=[
            pltpu.VMEM_SHARED((N_NODE, 128), _f32),
            pltpu.VMEM_SHARED((N_NODE, 16), _f32),
            pltpu.VMEM((ZROWS, 128), _f32),
            pltpu.VMEM((ZROWS, 16), _f32),
            pltpu.VMEM((W_EDGE,), jnp.int32),
            pltpu.VMEM((W_EDGE,), jnp.int32),
            pltpu.VMEM((W_EDGE,), jnp.int32),
            pltpu.VMEM((W_EDGE, 16), _f32),
            pltpu.VMEM((W_EDGE, 16), _f32),
            pltpu.VMEM((W_EDGE, 16), _f32),
            pltpu.VMEM((W_EDGE, 128), _f32),
        ],
    )
    def conv(h_au2_r, h_pa2_r, au_s_r, pa_s_r, srcw_r, dstw_r, srcr_r, dstr_r,
             ou_p, den_p, ou_a, den_a,
             acc, accd, zbuf, zdbuf, srcbuf, dstbuf, idx2buf,
             asrcbuf, adstbuf, exbuf, rowsbuf):
        sid = lax.axis_index("s")
        cid = lax.axis_index("c")

        zv = jnp.zeros((16,), _f32)

        @pl.loop(0, ZROWS)
        def _(r):
            for j in range(8):
                zbuf[r, pl.ds(j * 16, 16)] = zv
            zdbuf[r, :] = zv

        _sc_zero_acc(acc, accd, zbuf, zdbuf, sid)
        plsc.subcore_barrier()

        _sc_phase(0, h_au2_r, au_s_r, pa_s_r, srcw_r, dstw_r, acc, accd,
                  srcbuf, dstbuf, idx2buf, asrcbuf, adstbuf, exbuf, rowsbuf,
                  sid, cid)
        plsc.subcore_barrier()
        _sc_flush(acc, accd, ou_p, den_p, sid, cid)
        _sc_zero_acc(acc, accd, zbuf, zdbuf, sid)
        plsc.subcore_barrier()

        _sc_phase(1, h_pa2_r, pa_s_r, au_s_r, srcr_r, dstr_r, acc, accd,
                  srcbuf, dstbuf, idx2buf, asrcbuf, adstbuf, exbuf, rowsbuf,
                  sid, cid)
        plsc.subcore_barrier()
        _sc_flush(acc, accd, ou_a, den_a, sid, cid)

    return conv(h_au2, h_pa2, au_s, pa_s, src_w, dst_w, src_r, dst_r)


def _norm_body(oup0, oup1, dp0, dp1, oua0, oua1, da0, da1,
               rp0_ref, rp1_ref, ra0_ref, ra1_ref, wo_ref, bo_ref,
               out_ref, embp_ref, emba_ref):
    rexp_p = (jnp.dot(1.0 / (dp0[...] + 1e-16), rp0_ref[...],
                      preferred_element_type=_f32)
              + jnp.dot(1.0 / (dp1[...] + 1e-16), rp1_ref[...],
                        preferred_element_type=_f32))
    p_lo = jnp.maximum(oup0[...] * rexp_p[:, :128], 0.0)
    p_hi = jnp.maximum(oup1[...] * rexp_p[:, 128:], 0.0)
    embp_ref[:, 0:128] = p_lo
    embp_ref[:, 128:256] = p_hi

    rexp_a = (jnp.dot(1.0 / (da0[...] + 1e-16), ra0_ref[...],
                      preferred_element_type=_f32)
              + jnp.dot(1.0 / (da1[...] + 1e-16), ra1_ref[...],
                        preferred_element_type=_f32))
    emba_ref[:, 0:128] = jnp.maximum(oua0[...] * rexp_a[:, :128], 0.0)
    emba_ref[:, 128:256] = jnp.maximum(oua1[...] * rexp_a[:, 128:], 0.0)

    out_ref[...] = (jnp.dot(p_lo, wo_ref[0:128, :], preferred_element_type=_f32)
                    + jnp.dot(p_hi, wo_ref[128:256, :],
                              preferred_element_type=_f32)
                    + bo_ref[...])


def _normalize_head(ou_p, den_p, ou_a, den_a, w_out, b_out, rp0, rp1, ra0, ra1):
    blk = 1000
    grid = (N_NODE // blk,)
    node_spec = pl.BlockSpec((blk, 128), lambda i: (i, 0))
    den_spec = pl.BlockSpec((blk, 16), lambda i: (i, 0))
    r_spec = pl.BlockSpec((16, 256), lambda i: (0, 0))
    return pl.pallas_call(
        _norm_body,
        grid=grid,
        in_specs=[node_spec, node_spec, den_spec, den_spec,
                  node_spec, node_spec, den_spec, den_spec,
                  r_spec, r_spec, r_spec, r_spec,
                  pl.BlockSpec((D, D_OUT), lambda i: (0, 0)),
                  pl.BlockSpec((1, D_OUT), lambda i: (0, 0))],
        out_specs=[
            pl.BlockSpec((blk, D_OUT), lambda i: (i, 0)),
            pl.BlockSpec((blk, D), lambda i: (i, 0)),
            pl.BlockSpec((blk, D), lambda i: (i, 0)),
        ],
        out_shape=[
            jax.ShapeDtypeStruct((N_NODE, D_OUT), _f32),
            jax.ShapeDtypeStruct((N_NODE, D), _f32),
            jax.ShapeDtypeStruct((N_NODE, D), _f32),
        ],
    )(ou_p[0], ou_p[1], den_p[0], den_p[1],
      ou_a[0], ou_a[1], den_a[0], den_a[1],
      rp0, rp1, ra0, ra1, w_out, b_out)


_KR = np.kron(np.eye(N_HEADS), np.ones((D_HEAD, 1))).astype(np.float32)


def _r_mats():
    rp0 = np.zeros((16, 256), np.float32)
    rp1 = np.zeros((16, 256), np.float32)
    ra0 = np.zeros((16, 256), np.float32)
    ra1 = np.zeros((16, 256), np.float32)
    for h in range(4):
        rp0[h, h * 32:(h + 1) * 32] = 1.0
        ra0[8 + h, h * 32:(h + 1) * 32] = 1.0
    for h in range(4, 8):
        rp1[h, h * 32:(h + 1) * 32] = 1.0
        ra1[8 + h, h * 32:(h + 1) * 32] = 1.0
    return (jnp.asarray(rp0), jnp.asarray(rp1),
            jnp.asarray(ra0), jnp.asarray(ra1))


_RP0, _RP1, _RA0, _RA1 = _r_mats()


def kernel(x_author, x_paper, edge_index_writes, edge_index_rev,
           W_proj_author, b_proj_author, W_proj_paper, b_proj_paper,
           att_src_writes, att_dst_writes, att_src_rev, att_dst_rev,
           W_k_lin, b_k_lin, q_sem, W_out, b_out):
    kr = jnp.asarray(_KR)
    a_au = jnp.concatenate([att_src_writes.reshape(D, 1) * kr,
                            att_dst_rev.reshape(D, 1) * kr], axis=1)
    a_pa = jnp.concatenate([att_dst_writes.reshape(D, 1) * kr,
                            att_src_rev.reshape(D, 1) * kr], axis=1)

    h_au, au_s = _project(x_author, W_proj_author,
                          b_proj_author.reshape(1, D), a_au)
    h_pa, pa_s = _project(x_paper, W_proj_paper,
                          b_proj_paper.reshape(1, D), a_pa)

    ei_w = edge_index_writes.astype(jnp.int32)
    ei_r = edge_index_rev.astype(jnp.int32)
    ou_p, den_p, ou_a, den_a = _sc_conv(
        h_au.reshape(2 * N_NODE, 128), h_pa.reshape(2 * N_NODE, 128),
        au_s, pa_s, ei_w[0], ei_w[1], ei_r[0], ei_r[1])

    out, emb_p, emb_a = _normalize_head(
        ou_p, den_p, ou_a, den_a, W_out, b_out.reshape(1, D_OUT),
        _RP0, _RP1, _RA0, _RA1)
    return (out, emb_a, emb_p)


# speedup vs baseline: 20.6107x; 20.6107x over previous
import dataclasses
import functools

import jax
import jax.numpy as jnp
import numpy as np
from jax import lax
from jax.experimental import pallas as pl
from jax.experimental.pallas import tpu as pltpu
from jax.experimental.pallas import tpu_sc as plsc

N_NODE = 10000
N_EDGE = 160000
D = 256
N_HEADS = 8
D_HEAD = 32
D_OUT = 64

SC_CORES = 2
SC_TILES = 16
W_EDGE = 64
EDGES_PER_TILE = 10048
N_WIN = EDGES_PER_TILE // W_EDGE
E_PAD = EDGES_PER_TILE * SC_TILES
N_ABS = 48
N_ACC = N_NODE + N_ABS
ROWS_PER_TILE = N_NODE // SC_TILES
N_PAD = 10240

_f32 = jnp.float32


def _proj_body(x_ref, w_ref, b_ref, a_ref, h_ref, s_ref):
    h = jnp.dot(x_ref[...], w_ref[...], preferred_element_type=_f32)
    h = h + b_ref[...]
    h_ref[...] = h
    s_ref[...] = jnp.dot(h, a_ref[...], preferred_element_type=_f32)


def _project(x, w, b, a_mat):
    blk = 1000
    grid = (N_NODE // blk,)
    return pl.pallas_call(
        _proj_body,
        grid=grid,
        in_specs=[
            pl.BlockSpec((blk, D), lambda i: (i, 0)),
            pl.BlockSpec((D, D), lambda i: (0, 0)),
            pl.BlockSpec((1, D), lambda i: (0, 0)),
            pl.BlockSpec((D, 16), lambda i: (0, 0)),
        ],
        out_specs=[
            pl.BlockSpec((blk, D), lambda i: (i, 0)),
            pl.BlockSpec((blk, 16), lambda i: (i, 0)),
        ],
        out_shape=[
            jax.ShapeDtypeStruct((N_NODE, D), _f32),
            jax.ShapeDtypeStruct((N_NODE, 16), _f32),
        ],
    )(x, w, b, a_mat)


def _zero_bufs(abuf, dbuf):
    zv = jnp.zeros((16,), _f32)

    @pl.loop(0, 64)
    def _(r):
        for j in range(8):
            abuf[r, pl.ds(j * 16, 16)] = zv
        dbuf[r, :] = zv


def _sc_zero_acc(acc, accd, abuf, dbuf, sid):
    @pl.loop(0, 9)
    def _(c):
        base = sid * ROWS_PER_TILE + c * 64
        pltpu.sync_copy(abuf, acc.at[pl.ds(base, 64)])
        pltpu.sync_copy(dbuf, accd.at[pl.ds(base, 64)])

    tail = sid * ROWS_PER_TILE + 576
    pltpu.sync_copy(abuf.at[pl.ds(0, 49)], acc.at[pl.ds(tail, 49)])
    pltpu.sync_copy(dbuf.at[pl.ds(0, 49)], accd.at[pl.ds(tail, 49)])


_DNUMS = lax.GatherDimensionNumbers(
    offset_dims=(), collapsed_slice_dims=(0,), start_index_map=(0,))


def _bcast_lane(vec, lane):
    idx = jnp.full((16,), lane, jnp.int32)
    return lax.gather(vec, idx[:, None], _DNUMS, slice_sizes=(1,),
                      mode=lax.GatherScatterMode.PROMISE_IN_BOUNDS)


def _sc_phase(phase, h2, s_src_pk, s_dst_pk, src_hbm, dst_hbm, acc, accd,
              srcbuf, dstbuf, idx2buf, ridxs, ridxd, lbvs, lbvd,
              srows, drows, exbuf, rowsbuf, sid, cid):
    ex_base = phase * 8 + 4 * cid
    iota16 = jnp.arange(16, dtype=jnp.int32)

    @pl.loop(0, N_WIN)
    def _(w):
        off = sid * EDGES_PER_TILE + w * W_EDGE
        pltpu.sync_copy(src_hbm.at[pl.ds(off, W_EDGE)], srcbuf)
        pltpu.sync_copy(dst_hbm.at[pl.ds(off, W_EDGE)], dstbuf)

        @pl.loop(0, W_EDGE, step=16)
        def _(i):
            sv = srcbuf[pl.ds(i, 16)]
            dv = dstbuf[pl.ds(i, 16)]
            idx2buf[pl.ds(i, 16)] = sv + sv + cid
            ridxs[pl.ds(i, 16)] = lax.shift_right_logical(sv, 3)
            ridxd[pl.ds(i, 16)] = lax.shift_right_logical(dv, 3)
            lbvs[pl.ds(i, 16)] = (sv & 7) * 16
            lbvd[pl.ds(i, 16)] = (dv & 7) * 16

        pltpu.sync_copy(s_src_pk.at[ridxs], srows)
        pltpu.sync_copy(s_dst_pk.at[ridxd], drows)
        pltpu.sync_copy(h2.at[idx2buf], rowsbuf)

        @pl.loop(0, W_EDGE, step=16)
        def _(i):
            ls = lbvs[pl.ds(i, 16)]
            ld = lbvd[pl.ds(i, 16)]
            for l in range(16):
                e = i + l
                ev = jnp.full((16,), e, jnp.int32)
                vs = plsc.load_gather(srows, [ev, _bcast_lane(ls, l) + iota16])
                vd = plsc.load_gather(drows, [ev, _bcast_lane(ld, l) + iota16])
                av = vs + vd
                av = jnp.where(av > 0.0, av, 0.2 * av)
                exbuf[e, :] = jnp.exp(av)

        @pl.loop(0, W_EDGE)
        def _(e):
            ex_row = exbuf[e, :]
            for hh in range(4):
                scv = _bcast_lane(ex_row, ex_base + hh)
                for k in range(2):
                    sl = pl.ds(hh * 32 + k * 16, 16)
                    rowsbuf[e, sl] = rowsbuf[e, sl] * scv

        pltpu.sync_copy(rowsbuf, acc.at[dstbuf], add=True)
        pltpu.sync_copy(exbuf, accd.at[dstbuf], add=True)


def _norm_rows(ex_base, abuf, dbuf, nrows):
    @pl.loop(0, nrows)
    def _(r):
        rv = 1.0 / (dbuf[r, :] + 1e-16)
        for hh in range(4):
            ivv = _bcast_lane(rv, ex_base + hh)
            for k in range(2):
                sl = pl.ds(hh * 32 + k * 16, 16)
                abuf[r, sl] = jnp.maximum(abuf[r, sl] * ivv, 0.0)


def _sc_flush(phase, acc, accd, emb_hbm, abuf, dbuf, sid, cid):
    ex_base = phase * 8 + 4 * cid

    @pl.loop(0, 9)
    def _(c):
        b = sid * 624 + c * 64
        pltpu.sync_copy(acc.at[pl.ds(b, 64)], abuf)
        pltpu.sync_copy(accd.at[pl.ds(b, 64)], dbuf)
        _norm_rows(ex_base, abuf, dbuf, 64)
        pltpu.sync_copy(abuf, emb_hbm.at[cid].at[pl.ds(b, 64)])

    b48 = sid * 624 + 576
    pltpu.sync_copy(acc.at[pl.ds(b48, 48)], abuf.at[pl.ds(0, 48)])
    pltpu.sync_copy(accd.at[pl.ds(b48, 48)], dbuf.at[pl.ds(0, 48)])
    _norm_rows(ex_base, abuf, dbuf, 48)
    pltpu.sync_copy(abuf.at[pl.ds(0, 48)],
                    emb_hbm.at[cid].at[pl.ds(b48, 48)])

    @pl.when(sid == SC_TILES - 1)
    def _():
        pltpu.sync_copy(acc.at[pl.ds(9984, 16)], abuf.at[pl.ds(0, 16)])
        pltpu.sync_copy(accd.at[pl.ds(9984, 16)], dbuf.at[pl.ds(0, 16)])
        _norm_rows(ex_base, abuf, dbuf, 16)
        pltpu.sync_copy(abuf.at[pl.ds(0, 16)],
                        emb_hbm.at[cid].at[pl.ds(9984, 16)])


def _sc_conv(h_au2, h_pa2, au_s, pa_s, src_w, dst_w, src_r, dst_r):
    mesh = plsc.VectorSubcoreMesh(core_axis_name="c", subcore_axis_name="s")
    cp = pltpu.CompilerParams(needs_layout_passes=False,
                              use_tc_tiling_on_sc=False)

    @functools.partial(
        pl.kernel,
        mesh=mesh,
        compiler_params=cp,
        out_type=[
            jax.ShapeDtypeStruct((SC_CORES, N_NODE, 128), _f32),
            jax.ShapeDtypeStruct((SC_CORES, N_NODE, 128), _f32),
        ],
        scratch_types=[
            pltpu.VMEM_SHARED((N_ACC, 128), _f32),
            pltpu.VMEM_SHARED((N_ACC, 16), _f32),
            pltpu.VMEM((W_EDGE,), jnp.int32),
            pltpu.VMEM((W_EDGE,), jnp.int32),
            pltpu.VMEM((W_EDGE,), jnp.int32),
            pltpu.VMEM((W_EDGE,), jnp.int32),
            pltpu.VMEM((W_EDGE,), jnp.int32),
            pltpu.VMEM((W_EDGE,), jnp.int32),
            pltpu.VMEM((W_EDGE,), jnp.int32),
            pltpu.VMEM((W_EDGE, 128), _f32),
            pltpu.VMEM((W_EDGE, 128), _f32),
            pltpu.VMEM((W_EDGE, 16), _f32),
            pltpu.VMEM((W_EDGE, 128), _f32),
            pltpu.VMEM((64, 128), _f32),
            pltpu.VMEM((64, 16), _f32),
        ],
    )
    def conv(h_au2_r, h_pa2_r, au_s_r, pa_s_r, srcw_r, dstw_r, srcr_r, dstr_r,
             emb_p, emb_a,
             acc, accd, srcbuf, dstbuf, idx2buf,
             ridxs, ridxd, lbvs, lbvd, srows, drows, exbuf, rowsbuf,
             abuf, dbuf):
        sid = lax.axis_index("s")
        cid = lax.axis_index("c")

        _zero_bufs(abuf, dbuf)
        _sc_zero_acc(acc, accd, abuf, dbuf, sid)
        plsc.subcore_barrier()

        _sc_phase(0, h_au2_r, au_s_r, pa_s_r, srcw_r, dstw_r, acc, accd,
                  srcbuf, dstbuf, idx2buf, ridxs, ridxd, lbvs, lbvd,
                  srows, drows, exbuf, rowsbuf, sid, cid)
        plsc.subcore_barrier()
        _sc_flush(0, acc, accd, emb_p, abuf, dbuf, sid, cid)
        _zero_bufs(abuf, dbuf)
        _sc_zero_acc(acc, accd, abuf, dbuf, sid)
        plsc.subcore_barrier()

        _sc_phase(1, h_pa2_r, pa_s_r, au_s_r, srcr_r, dstr_r, acc, accd,
                  srcbuf, dstbuf, idx2buf, ridxs, ridxd, lbvs, lbvd,
                  srows, drows, exbuf, rowsbuf, sid, cid)
        plsc.subcore_barrier()
        _sc_flush(1, acc, accd, emb_a, abuf, dbuf, sid, cid)

    return conv(h_au2, h_pa2, au_s, pa_s, src_w, dst_w, src_r, dst_r)


def _head_body(embp0, embp1, emba0, emba1, wo_ref, bo_ref,
               out_ref, embp_ref, emba_ref):
    p_lo = embp0[...]
    p_hi = embp1[...]
    embp_ref[:, 0:128] = p_lo
    embp_ref[:, 128:256] = p_hi
    emba_ref[:, 0:128] = emba0[...]
    emba_ref[:, 128:256] = emba1[...]
    out_ref[...] = (jnp.dot(p_lo, wo_ref[0:128, :], preferred_element_type=_f32)
                    + jnp.dot(p_hi, wo_ref[128:256, :],
                              preferred_element_type=_f32)
                    + bo_ref[...])


def _assemble_head(emb_p2, emb_a2, w_out, b_out):
    blk = 1000
    grid = (N_NODE // blk,)
    node_spec = pl.BlockSpec((blk, 128), lambda i: (i, 0))
    return pl.pallas_call(
        _head_body,
        grid=grid,
        in_specs=[node_spec, node_spec, node_spec, node_spec,
                  pl.BlockSpec((D, D_OUT), lambda i: (0, 0)),
                  pl.BlockSpec((1, D_OUT), lambda i: (0, 0))],
        out_specs=[
            pl.BlockSpec((blk, D_OUT), lambda i: (i, 0)),
            pl.BlockSpec((blk, D), lambda i: (i, 0)),
            pl.BlockSpec((blk, D), lambda i: (i, 0)),
        ],
        out_shape=[
            jax.ShapeDtypeStruct((N_NODE, D_OUT), _f32),
            jax.ShapeDtypeStruct((N_NODE, D), _f32),
            jax.ShapeDtypeStruct((N_NODE, D), _f32),
        ],
    )(emb_p2[0], emb_p2[1], emb_a2[0], emb_a2[1], w_out, b_out)


_KR = np.kron(np.eye(N_HEADS), np.ones((D_HEAD, 1))).astype(np.float32)


def kernel(x_author, x_paper, edge_index_writes, edge_index_rev,
           W_proj_author, b_proj_author, W_proj_paper, b_proj_paper,
           att_src_writes, att_dst_writes, att_src_rev, att_dst_rev,
           W_k_lin, b_k_lin, q_sem, W_out, b_out):
    kr = jnp.asarray(_KR)
    a_au = jnp.concatenate([att_src_writes.reshape(D, 1) * kr,
                            att_dst_rev.reshape(D, 1) * kr], axis=1)
    a_pa = jnp.concatenate([att_dst_writes.reshape(D, 1) * kr,
                            att_src_rev.reshape(D, 1) * kr], axis=1)

    h_au, au_s = _project(x_author, W_proj_author,
                          b_proj_author.reshape(1, D), a_au)
    h_pa, pa_s = _project(x_paper, W_proj_paper,
                          b_proj_paper.reshape(1, D), a_pa)

    n_extra = E_PAD - N_EDGE
    pad_src = jnp.asarray((np.arange(n_extra) * 131) % N_NODE, jnp.int32)
    pad_dst = jnp.asarray(N_NODE + (np.arange(n_extra) % N_ABS), jnp.int32)

    def _pad_edges(ei):
        ei = ei.astype(jnp.int32)
        return (jnp.concatenate([ei[0], pad_src]),
                jnp.concatenate([ei[1], pad_dst]))

    ei_w = _pad_edges(edge_index_writes)
    ei_r = _pad_edges(edge_index_rev)
    au_s = jnp.pad(au_s, ((0, N_PAD - N_NODE), (0, 0))).reshape(N_PAD // 8, 128)
    pa_s = jnp.pad(pa_s, ((0, N_PAD - N_NODE), (0, 0))).reshape(N_PAD // 8, 128)
    emb_p2, emb_a2 = _sc_conv(
        h_au.reshape(2 * N_NODE, 128), h_pa.reshape(2 * N_NODE, 128),
        au_s, pa_s, ei_w[0], ei_w[1], ei_r[0], ei_r[1])

    out, emb_p, emb_a = _assemble_head(emb_p2, emb_a2, W_out,
                                       b_out.reshape(1, D_OUT))
    return (out, emb_a, emb_p)


# speedup vs baseline: 45.9355x; 2.2287x over previous
import dataclasses
import functools

import jax
import jax.numpy as jnp
import numpy as np
from jax import lax
from jax.experimental import pallas as pl
from jax.experimental.pallas import tpu as pltpu
from jax.experimental.pallas import tpu_sc as plsc

N_NODE = 10000
N_EDGE = 160000
D = 256
N_HEADS = 8
D_HEAD = 32
D_OUT = 64

SC_CORES = 2
SC_TILES = 16
W_EDGE = 48
EDGES_PER_TILE = 10080
N_WIN = EDGES_PER_TILE // W_EDGE
E_PAD = EDGES_PER_TILE * SC_TILES
N_ABS = 48
N_ACC = N_NODE + N_ABS
ROWS_PER_TILE = N_NODE // SC_TILES
N_PAD = 10240

_f32 = jnp.float32


def _proj_body(x_ref, w_ref, b_ref, a_ref, h_ref, s_ref):
    h = jnp.dot(x_ref[...], w_ref[...], preferred_element_type=_f32)
    h = h + b_ref[...]
    h_ref[...] = h
    s_ref[...] = jnp.dot(h, a_ref[...], preferred_element_type=_f32)


def _project(x, w, b, a_mat):
    blk = 1000
    grid = (N_NODE // blk,)
    return pl.pallas_call(
        _proj_body,
        grid=grid,
        in_specs=[
            pl.BlockSpec((blk, D), lambda i: (i, 0)),
            pl.BlockSpec((D, D), lambda i: (0, 0)),
            pl.BlockSpec((1, D), lambda i: (0, 0)),
            pl.BlockSpec((D, 16), lambda i: (0, 0)),
        ],
        out_specs=[
            pl.BlockSpec((blk, D), lambda i: (i, 0)),
            pl.BlockSpec((blk, 16), lambda i: (i, 0)),
        ],
        out_shape=[
            jax.ShapeDtypeStruct((N_NODE, D), _f32),
            jax.ShapeDtypeStruct((N_NODE, 16), _f32),
        ],
    )(x, w, b, a_mat)


def _zero_bufs(abuf, dbuf):
    zv = jnp.zeros((16,), _f32)

    @pl.loop(0, 48)
    def _(r):
        for j in range(8):
            abuf[r, pl.ds(j * 16, 16)] = zv
        dbuf[r, :] = zv


def _sc_zero_acc(acc, accd, abuf, dbuf, sid):
    @pl.loop(0, 13)
    def _(c):
        base = sid * ROWS_PER_TILE + c * 48
        pltpu.sync_copy(abuf, acc.at[pl.ds(base, 48)])
        pltpu.sync_copy(dbuf, accd.at[pl.ds(base, 48)])

    tail = sid * ROWS_PER_TILE + 624
    pltpu.sync_copy(abuf.at[pl.ds(0, 1)], acc.at[pl.ds(tail, 1)])
    pltpu.sync_copy(dbuf.at[pl.ds(0, 1)], accd.at[pl.ds(tail, 1)])


_DNUMS = lax.GatherDimensionNumbers(
    offset_dims=(), collapsed_slice_dims=(0,), start_index_map=(0,))


def _bcast_lane(vec, lane):
    idx = jnp.full((16,), lane, jnp.int32)
    return lax.gather(vec, idx[:, None], _DNUMS, slice_sizes=(1,),
                      mode=lax.GatherScatterMode.PROMISE_IN_BOUNDS)


def _sc_phase(phase, h2, s_src_pk, s_dst_pk, src_hbm, dst_hbm, acc, accd,
              bufs, sems, sid, cid):
    ex_base = phase * 8 + 4 * cid
    iota16 = jnp.arange(16, dtype=jnp.int32)
    base_e = sid * EDGES_PER_TILE
    sem_i, sem_g, sem_s = sems

    def idx_descr(w, p):
        off = base_e + w * W_EDGE
        return (
            pltpu.make_async_copy(src_hbm.at[pl.ds(off, W_EDGE)],
                                  bufs["src"][p], sem_i[p]),
            pltpu.make_async_copy(dst_hbm.at[pl.ds(off, W_EDGE)],
                                  bufs["dst"][p], sem_i[p]),
        )

    def start_idx(w, p):
        for d in idx_descr(w, p):
            d.start()

    def wait_idx(w, p):
        for d in idx_descr(w, p):
            d.wait()

    def transforms(p):
        @pl.loop(0, W_EDGE, step=16)
        def _(i):
            sv = bufs["src"][p][pl.ds(i, 16)]
            bufs["idx2"][p][pl.ds(i, 16)] = sv + sv + cid

    def g_descr(p):
        return (
            pltpu.make_async_copy(s_src_pk.at[bufs["src"][p]],
                                  bufs["srw"][p], sem_g[p]),
            pltpu.make_async_copy(s_dst_pk.at[bufs["dst"][p]],
                                  bufs["drw"][p], sem_g[p]),
            pltpu.make_async_copy(h2.at[bufs["idx2"][p]],
                                  bufs["rwb"][p], sem_g[p]),
        )

    def start_gathers(p):
        for d in g_descr(p):
            d.start()

    def wait_gathers(p):
        for d in g_descr(p):
            d.wait()

    def compute(p):
        srows, drows = bufs["srw"][p], bufs["drw"][p]
        exbuf, rowsbuf = bufs["exb"][p], bufs["rwb"][p]

        @pl.loop(0, W_EDGE)
        def _(e):
            av = srows[e, :] + drows[e, :]
            av = jnp.where(av > 0.0, av, 0.2 * av)
            exbuf[e, :] = jnp.exp(av)

        @pl.loop(0, W_EDGE, step=16)
        def _(i):
            bufs["dsc"][p][pl.ds(i, 16)] = bufs["dst"][p][pl.ds(i, 16)]

        @pl.loop(0, W_EDGE)
        def _(e):
            ex_row = exbuf[e, :]
            for hh in range(4):
                scv = _bcast_lane(ex_row, ex_base + hh)
                for k in range(2):
                    sl = pl.ds(hh * 32 + k * 16, 16)
                    rowsbuf[e, sl] = rowsbuf[e, sl] * scv

    def s_descr(p):
        return (
            pltpu.make_async_copy(bufs["rwb"][p], acc.at[bufs["dsc"][p]],
                                  sem_s[p]),
            pltpu.make_async_copy(bufs["exb"][p], accd.at[bufs["dsc"][p]],
                                  sem_s[p]),
        )

    def start_scatters(p):
        for d in s_descr(p):
            d.start(add=True)

    def wait_scatters(p):
        for d in s_descr(p):
            d.wait()

    def steady(w, p, last_gather, last_idx):
        q = 1 - p
        wait_gathers(p)
        compute(p)
        start_scatters(p)
        wait_scatters(q)
        if not last_gather:
            wait_idx(w + 1, q)
            transforms(q)
            start_gathers(q)
            if not last_idx:
                start_idx(w + 2, p)

    start_idx(0, 0)
    wait_idx(0, 0)
    transforms(0)
    start_gathers(0)
    start_idx(1, 1)
    wait_gathers(0)
    compute(0)
    start_scatters(0)
    wait_idx(1, 1)
    transforms(1)
    start_gathers(1)
    start_idx(2, 0)

    @pl.loop(0, (N_WIN - 4) // 2)
    def _(k):
        steady(2 * k + 1, 1, False, False)
        steady(2 * k + 2, 0, False, False)

    steady(N_WIN - 3, 1, False, False)
    steady(N_WIN - 2, 0, False, True)
    steady(N_WIN - 1, 1, True, True)
    wait_scatters(1)


def _norm_rows(ex_base, abuf, dbuf, nrows):
    @pl.loop(0, nrows)
    def _(r):
        rv = 1.0 / (dbuf[r, :] + 1e-16)
        for hh in range(4):
            ivv = _bcast_lane(rv, ex_base + hh)
            for k in range(2):
                sl = pl.ds(hh * 32 + k * 16, 16)
                abuf[r, sl] = jnp.maximum(abuf[r, sl] * ivv, 0.0)


def _sc_flush(phase, acc, accd, emb_hbm, abuf, dbuf, sid, cid):
    ex_base = phase * 8 + 4 * cid

    @pl.loop(0, 13)
    def _(c):
        b = sid * 624 + c * 48
        pltpu.sync_copy(acc.at[pl.ds(b, 48)], abuf)
        pltpu.sync_copy(accd.at[pl.ds(b, 48)], dbuf)
        _norm_rows(ex_base, abuf, dbuf, 48)
        pltpu.sync_copy(abuf, emb_hbm.at[cid].at[pl.ds(b, 48)])

    @pl.when(sid == SC_TILES - 1)
    def _():
        pltpu.sync_copy(acc.at[pl.ds(9984, 16)], abuf.at[pl.ds(0, 16)])
        pltpu.sync_copy(accd.at[pl.ds(9984, 16)], dbuf.at[pl.ds(0, 16)])
        _norm_rows(ex_base, abuf, dbuf, 16)
        pltpu.sync_copy(abuf.at[pl.ds(0, 16)],
                        emb_hbm.at[cid].at[pl.ds(9984, 16)])


def _sc_conv(h_au2, h_pa2, au_s, pa_s, src_w, dst_w, src_r, dst_r):
    mesh = plsc.VectorSubcoreMesh(core_axis_name="c", subcore_axis_name="s")
    cp = pltpu.CompilerParams(needs_layout_passes=False,
                              use_tc_tiling_on_sc=False)

    @functools.partial(
        pl.kernel,
        mesh=mesh,
        compiler_params=cp,
        out_type=[
            jax.ShapeDtypeStruct((SC_CORES, N_NODE, 128), _f32),
            jax.ShapeDtypeStruct((SC_CORES, N_NODE, 128), _f32),
        ],
        scratch_types=(
            [pltpu.VMEM_SHARED((N_ACC, 128), _f32),
             pltpu.VMEM_SHARED((N_ACC, 16), _f32)]
            + [pltpu.VMEM((W_EDGE,), jnp.int32)] * 8
            + [pltpu.VMEM((W_EDGE, 16), _f32)] * 4
            + [pltpu.VMEM((W_EDGE, 128), _f32)] * 2
            + [pltpu.VMEM((W_EDGE, 16), _f32)] * 2
            + [pltpu.VMEM((48, 128), _f32),
               pltpu.VMEM((48, 16), _f32)]
            + [pltpu.SemaphoreType.DMA] * 6
        ),
    )
    def conv(h_au2_r, h_pa2_r, au_s_r, pa_s_r, srcw_r, dstw_r, srcr_r, dstr_r,
             emb_p, emb_a,
             acc, accd,
             src0, src1, dst0, dst1, idx20, idx21, dsc0, dsc1,
             srw0, srw1, drw0, drw1, rwb0, rwb1, exb0, exb1,
             abuf, dbuf,
             semi0, semi1, semg0, semg1, sems0, sems1):
        sid = lax.axis_index("s")
        cid = lax.axis_index("c")
        bufs = {
            "src": (src0, src1), "dst": (dst0, dst1),
            "idx2": (idx20, idx21), "dsc": (dsc0, dsc1),
            "srw": (srw0, srw1), "drw": (drw0, drw1), "rwb": (rwb0, rwb1),
            "exb": (exb0, exb1),
        }
        sems = ((semi0, semi1), (semg0, semg1), (sems0, sems1))

        _zero_bufs(abuf, dbuf)
        _sc_zero_acc(acc, accd, abuf, dbuf, sid)
        plsc.subcore_barrier()

        _sc_phase(0, h_au2_r, au_s_r, pa_s_r, srcw_r, dstw_r, acc, accd,
                  bufs, sems, sid, cid)
        plsc.subcore_barrier()
        _sc_flush(0, acc, accd, emb_p, abuf, dbuf, sid, cid)
        _zero_bufs(abuf, dbuf)
        _sc_zero_acc(acc, accd, abuf, dbuf, sid)
        plsc.subcore_barrier()

        _sc_phase(1, h_pa2_r, pa_s_r, au_s_r, srcr_r, dstr_r, acc, accd,
                  bufs, sems, sid, cid)
        plsc.subcore_barrier()
        _sc_flush(1, acc, accd, emb_a, abuf, dbuf, sid, cid)

    return conv(h_au2, h_pa2, au_s, pa_s, src_w, dst_w, src_r, dst_r)


def _head_body(embp0, embp1, emba0, emba1, wo_ref, bo_ref,
               out_ref, embp_ref, emba_ref):
    p_lo = embp0[...]
    p_hi = embp1[...]
    embp_ref[:, 0:128] = p_lo
    embp_ref[:, 128:256] = p_hi
    emba_ref[:, 0:128] = emba0[...]
    emba_ref[:, 128:256] = emba1[...]
    out_ref[...] = (jnp.dot(p_lo, wo_ref[0:128, :], preferred_element_type=_f32)
                    + jnp.dot(p_hi, wo_ref[128:256, :],
                              preferred_element_type=_f32)
                    + bo_ref[...])


def _assemble_head(emb_p2, emb_a2, w_out, b_out):
    blk = 1000
    grid = (N_NODE // blk,)
    node_spec = pl.BlockSpec((blk, 128), lambda i: (i, 0))
    return pl.pallas_call(
        _head_body,
        grid=grid,
        in_specs=[node_spec, node_spec, node_spec, node_spec,
                  pl.BlockSpec((D, D_OUT), lambda i: (0, 0)),
                  pl.BlockSpec((1, D_OUT), lambda i: (0, 0))],
        out_specs=[
            pl.BlockSpec((blk, D_OUT), lambda i: (i, 0)),
            pl.BlockSpec((blk, D), lambda i: (i, 0)),
            pl.BlockSpec((blk, D), lambda i: (i, 0)),
        ],
        out_shape=[
            jax.ShapeDtypeStruct((N_NODE, D_OUT), _f32),
            jax.ShapeDtypeStruct((N_NODE, D), _f32),
            jax.ShapeDtypeStruct((N_NODE, D), _f32),
        ],
    )(emb_p2[0], emb_p2[1], emb_a2[0], emb_a2[1], w_out, b_out)


_KR = np.kron(np.eye(N_HEADS), np.ones((D_HEAD, 1))).astype(np.float32)


def kernel(x_author, x_paper, edge_index_writes, edge_index_rev,
           W_proj_author, b_proj_author, W_proj_paper, b_proj_paper,
           att_src_writes, att_dst_writes, att_src_rev, att_dst_rev,
           W_k_lin, b_k_lin, q_sem, W_out, b_out):
    kr = jnp.asarray(_KR)
    a_au = jnp.concatenate([att_src_writes.reshape(D, 1) * kr,
                            att_dst_rev.reshape(D, 1) * kr], axis=1)
    a_pa = jnp.concatenate([att_dst_writes.reshape(D, 1) * kr,
                            att_src_rev.reshape(D, 1) * kr], axis=1)

    h_au, au_s = _project(x_author, W_proj_author,
                          b_proj_author.reshape(1, D), a_au)
    h_pa, pa_s = _project(x_paper, W_proj_paper,
                          b_proj_paper.reshape(1, D), a_pa)

    n_extra = E_PAD - N_EDGE
    pad_src = jnp.asarray((np.arange(n_extra) * 131) % N_NODE, jnp.int32)
    pad_dst = jnp.asarray(N_NODE + (np.arange(n_extra) % N_ABS), jnp.int32)

    def _pad_edges(ei):
        ei = ei.astype(jnp.int32)
        return (jnp.concatenate([ei[0], pad_src]),
                jnp.concatenate([ei[1], pad_dst]))

    ei_w = _pad_edges(edge_index_writes)
    ei_r = _pad_edges(edge_index_rev)
    au_s = jnp.pad(au_s, ((0, N_PAD - N_NODE), (0, 0)))
    pa_s = jnp.pad(pa_s, ((0, N_PAD - N_NODE), (0, 0)))
    emb_p2, emb_a2 = _sc_conv(
        h_au.reshape(2 * N_NODE, 128), h_pa.reshape(2 * N_NODE, 128),
        au_s, pa_s, ei_w[0], ei_w[1], ei_r[0], ei_r[1])

    out, emb_p, emb_a = _assemble_head(emb_p2, emb_a2, W_out,
                                       b_out.reshape(1, D_OUT))
    return (out, emb_a, emb_p)


# speedup vs baseline: 53.2499x; 1.1592x over previous
import dataclasses
import functools

import jax
import jax.numpy as jnp
import numpy as np
from jax import lax
from jax.experimental import pallas as pl
from jax.experimental.pallas import tpu as pltpu
from jax.experimental.pallas import tpu_sc as plsc

N_NODE = 10000
N_EDGE = 160000
D = 256
N_HEADS = 8
D_HEAD = 32
D_OUT = 64

SC_CORES = 2
SC_TILES = 16
W_EDGE = 96
EDGES_PER_TILE = 10176
N_WIN = EDGES_PER_TILE // W_EDGE
E_PAD = EDGES_PER_TILE * SC_TILES
N_ABS = 16
N_ACC = N_NODE + N_ABS
ROWS_PER_TILE = N_NODE // SC_TILES
N_PAD = 10240

_f32 = jnp.float32


def _proj_body(x_ref, w_ref, b_ref, a_ref, h_ref, s_ref):
    h = jnp.dot(x_ref[...], w_ref[...], preferred_element_type=_f32)
    h = h + b_ref[...]
    h_ref[...] = h
    s_ref[...] = jnp.dot(h, a_ref[...], preferred_element_type=_f32)


def _project(x, w, b, a_mat):
    blk = 1000
    grid = (N_NODE // blk,)
    return pl.pallas_call(
        _proj_body,
        grid=grid,
        in_specs=[
            pl.BlockSpec((blk, D), lambda i: (i, 0)),
            pl.BlockSpec((D, D), lambda i: (0, 0)),
            pl.BlockSpec((1, D), lambda i: (0, 0)),
            pl.BlockSpec((D, 16), lambda i: (0, 0)),
        ],
        out_specs=[
            pl.BlockSpec((blk, D), lambda i: (i, 0)),
            pl.BlockSpec((blk, 16), lambda i: (i, 0)),
        ],
        out_shape=[
            jax.ShapeDtypeStruct((N_NODE, D), _f32),
            jax.ShapeDtypeStruct((N_NODE, 16), _f32),
        ],
    )(x, w, b, a_mat)


def _zero_bufs(abuf, dbuf):
    zv = jnp.zeros((16,), _f32)

    @pl.loop(0, 32)
    def _(r):
        for j in range(8):
            abuf[r, pl.ds(j * 16, 16)] = zv
        dbuf[r, :] = zv


def _sc_zero_acc(acc, accd, abuf, dbuf, sid):
    @pl.loop(0, 19)
    def _(c):
        base = sid * ROWS_PER_TILE + c * 32
        pltpu.sync_copy(abuf, acc.at[pl.ds(base, 32)])
        pltpu.sync_copy(dbuf, accd.at[pl.ds(base, 32)])

    tail = sid * ROWS_PER_TILE + 608
    pltpu.sync_copy(abuf.at[pl.ds(0, 17)], acc.at[pl.ds(tail, 17)])
    pltpu.sync_copy(dbuf.at[pl.ds(0, 17)], accd.at[pl.ds(tail, 17)])


_DNUMS = lax.GatherDimensionNumbers(
    offset_dims=(), collapsed_slice_dims=(0,), start_index_map=(0,))


def _bcast_lane(vec, lane):
    idx = jnp.full((16,), lane, jnp.int32)
    return lax.gather(vec, idx[:, None], _DNUMS, slice_sizes=(1,),
                      mode=lax.GatherScatterMode.PROMISE_IN_BOUNDS)


def _sc_phase(phase, h2, s_src_pk, s_dst_pk, src_hbm, dst_hbm, acc, accd,
              bufs, sems, sid, cid):
    ex_base = phase * 8 + 4 * cid
    iota16 = jnp.arange(16, dtype=jnp.int32)
    base_e = sid * EDGES_PER_TILE
    sem_i, sem_g, sem_s = sems

    def idx_descr(w, p):
        off = base_e + w * W_EDGE
        return (
            pltpu.make_async_copy(src_hbm.at[pl.ds(off, W_EDGE)],
                                  bufs["src"][p], sem_i[p]),
            pltpu.make_async_copy(dst_hbm.at[pl.ds(off, W_EDGE)],
                                  bufs["dst"][p], sem_i[p]),
        )

    def start_idx(w, p):
        for d in idx_descr(w, p):
            d.start()

    def wait_idx(w, p):
        for d in idx_descr(w, p):
            d.wait()

    def transforms(p):
        @pl.loop(0, W_EDGE, step=16)
        def _(i):
            sv = bufs["src"][p][pl.ds(i, 16)]
            bufs["idx2"][p][pl.ds(i, 16)] = sv + sv + cid

    def g_descr(p):
        return (
            pltpu.make_async_copy(s_src_pk.at[bufs["src"][p]],
                                  bufs["srw"][p], sem_g[p]),
            pltpu.make_async_copy(s_dst_pk.at[bufs["dst"][p]],
                                  bufs["drw"][p], sem_g[p]),
            pltpu.make_async_copy(h2.at[bufs["idx2"][p]],
                                  bufs["rwb"][p], sem_g[p]),
        )

    def start_gathers(p):
        for d in g_descr(p):
            d.start()

    def wait_gathers(p):
        for d in g_descr(p):
            d.wait()

    def compute(p):
        srows, drows = bufs["srw"][p], bufs["drw"][p]
        exbuf, rowsbuf = bufs["exb"][p], bufs["rwb"][p]

        @pl.loop(0, W_EDGE)
        def _(e):
            av = srows[e, :] + drows[e, :]
            av = jnp.where(av > 0.0, av, 0.2 * av)
            exbuf[e, :] = jnp.exp(av)

        @pl.loop(0, W_EDGE, step=16)
        def _(i):
            bufs["dsc"][p][pl.ds(i, 16)] = bufs["dst"][p][pl.ds(i, 16)]

        @pl.loop(0, W_EDGE)
        def _(e):
            ex_row = exbuf[e, :]
            for hh in range(4):
                scv = _bcast_lane(ex_row, ex_base + hh)
                for k in range(2):
                    sl = pl.ds(hh * 32 + k * 16, 16)
                    rowsbuf[e, sl] = rowsbuf[e, sl] * scv

    def s_descr(p):
        return (
            pltpu.make_async_copy(bufs["rwb"][p], acc.at[bufs["dsc"][p]],
                                  sem_s[p]),
            pltpu.make_async_copy(bufs["exb"][p], accd.at[bufs["dsc"][p]],
                                  sem_s[p]),
        )

    def start_scatters(p):
        for d in s_descr(p):
            d.start(add=True)

    def wait_scatters(p):
        for d in s_descr(p):
            d.wait()

    def steady(w, p, last_gather, last_idx):
        q = 1 - p
        wait_gathers(p)
        compute(p)
        start_scatters(p)
        wait_scatters(q)
        if not last_gather:
            wait_idx(w + 1, q)
            transforms(q)
            start_gathers(q)
            if not last_idx:
                start_idx(w + 2, p)

    start_idx(0, 0)
    wait_idx(0, 0)
    transforms(0)
    start_gathers(0)
    start_idx(1, 1)
    wait_gathers(0)
    compute(0)
    start_scatters(0)
    wait_idx(1, 1)
    transforms(1)
    start_gathers(1)
    start_idx(2, 0)

    @pl.loop(0, (N_WIN - 4) // 2)
    def _(k):
        steady(2 * k + 1, 1, False, False)
        steady(2 * k + 2, 0, False, False)

    steady(N_WIN - 3, 1, False, False)
    steady(N_WIN - 2, 0, False, True)
    steady(N_WIN - 1, 1, True, True)
    wait_scatters(1)


def _norm_rows(ex_base, abuf, dbuf, nrows):
    @pl.loop(0, nrows)
    def _(r):
        rv = 1.0 / (dbuf[r, :] + 1e-16)
        for hh in range(4):
            ivv = _bcast_lane(rv, ex_base + hh)
            for k in range(2):
                sl = pl.ds(hh * 32 + k * 16, 16)
                abuf[r, sl] = jnp.maximum(abuf[r, sl] * ivv, 0.0)


def _sc_flush(phase, acc, accd, emb_hbm, abuf, dbuf, sid, cid):
    ex_base = phase * 8 + 4 * cid

    @pl.loop(0, 19)
    def _(c):
        b = sid * 624 + c * 32
        pltpu.sync_copy(acc.at[pl.ds(b, 32)], abuf)
        pltpu.sync_copy(accd.at[pl.ds(b, 32)], dbuf)
        _norm_rows(ex_base, abuf, dbuf, 32)
        pltpu.sync_copy(abuf, emb_hbm.at[cid].at[pl.ds(b, 32)])

    b16 = sid * 624 + 608
    pltpu.sync_copy(acc.at[pl.ds(b16, 16)], abuf.at[pl.ds(0, 16)])
    pltpu.sync_copy(accd.at[pl.ds(b16, 16)], dbuf.at[pl.ds(0, 16)])
    _norm_rows(ex_base, abuf, dbuf, 16)
    pltpu.sync_copy(abuf.at[pl.ds(0, 16)],
                    emb_hbm.at[cid].at[pl.ds(b16, 16)])

    @pl.when(sid == SC_TILES - 1)
    def _():
        pltpu.sync_copy(acc.at[pl.ds(9984, 16)], abuf.at[pl.ds(0, 16)])
        pltpu.sync_copy(accd.at[pl.ds(9984, 16)], dbuf.at[pl.ds(0, 16)])
        _norm_rows(ex_base, abuf, dbuf, 16)
        pltpu.sync_copy(abuf.at[pl.ds(0, 16)],
                        emb_hbm.at[cid].at[pl.ds(9984, 16)])


def _sc_conv(h_au2, h_pa2, au_s, pa_s, src_w, dst_w, src_r, dst_r):
    mesh = plsc.VectorSubcoreMesh(core_axis_name="c", subcore_axis_name="s")
    cp = pltpu.CompilerParams(needs_layout_passes=False,
                              use_tc_tiling_on_sc=False)

    @functools.partial(
        pl.kernel,
        mesh=mesh,
        compiler_params=cp,
        out_type=[
            jax.ShapeDtypeStruct((SC_CORES, N_NODE, 128), _f32),
            jax.ShapeDtypeStruct((SC_CORES, N_NODE, 128), _f32),
        ],
        scratch_types=(
            [pltpu.VMEM_SHARED((N_ACC, 128), _f32),
             pltpu.VMEM_SHARED((N_ACC, 16), _f32)]
            + [pltpu.VMEM((W_EDGE,), jnp.int32)] * 8
            + [pltpu.VMEM((W_EDGE, 16), _f32)] * 4
            + [pltpu.VMEM((W_EDGE, 128), _f32)] * 2
            + [pltpu.VMEM((W_EDGE, 16), _f32)] * 2
            + [pltpu.VMEM((32, 128), _f32),
               pltpu.VMEM((32, 16), _f32)]
            + [pltpu.SemaphoreType.DMA] * 6
        ),
    )
    def conv(h_au2_r, h_pa2_r, au_s_r, pa_s_r, srcw_r, dstw_r, srcr_r, dstr_r,
             emb_p, emb_a,
             acc, accd,
             src0, src1, dst0, dst1, idx20, idx21, dsc0, dsc1,
             srw0, srw1, drw0, drw1, rwb0, rwb1, exb0, exb1,
             abuf, dbuf,
             semi0, semi1, semg0, semg1, sems0, sems1):
        sid = lax.axis_index("s")
        cid = lax.axis_index("c")
        bufs = {
            "src": (src0, src1), "dst": (dst0, dst1),
            "idx2": (idx20, idx21), "dsc": (dsc0, dsc1),
            "srw": (srw0, srw1), "drw": (drw0, drw1), "rwb": (rwb0, rwb1),
            "exb": (exb0, exb1),
        }
        sems = ((semi0, semi1), (semg0, semg1), (sems0, sems1))

        _zero_bufs(abuf, dbuf)
        _sc_zero_acc(acc, accd, abuf, dbuf, sid)
        plsc.subcore_barrier()

        _sc_phase(0, h_au2_r, au_s_r, pa_s_r, srcw_r, dstw_r, acc, accd,
                  bufs, sems, sid, cid)
        plsc.subcore_barrier()
        _sc_flush(0, acc, accd, emb_p, abuf, dbuf, sid, cid)
        _zero_bufs(abuf, dbuf)
        _sc_zero_acc(acc, accd, abuf, dbuf, sid)
        plsc.subcore_barrier()

        _sc_phase(1, h_pa2_r, pa_s_r, au_s_r, srcr_r, dstr_r, acc, accd,
                  bufs, sems, sid, cid)
        plsc.subcore_barrier()
        _sc_flush(1, acc, accd, emb_a, abuf, dbuf, sid, cid)

    return conv(h_au2, h_pa2, au_s, pa_s, src_w, dst_w, src_r, dst_r)


def _head_body(embp0, embp1, emba0, emba1, wo_ref, bo_ref,
               out_ref, embp_ref, emba_ref):
    p_lo = embp0[...]
    p_hi = embp1[...]
    embp_ref[:, 0:128] = p_lo
    embp_ref[:, 128:256] = p_hi
    emba_ref[:, 0:128] = emba0[...]
    emba_ref[:, 128:256] = emba1[...]
    out_ref[...] = (jnp.dot(p_lo, wo_ref[0:128, :], preferred_element_type=_f32)
                    + jnp.dot(p_hi, wo_ref[128:256, :],
                              preferred_element_type=_f32)
                    + bo_ref[...])


def _assemble_head(emb_p2, emb_a2, w_out, b_out):
    blk = 1000
    grid = (N_NODE // blk,)
    node_spec = pl.BlockSpec((blk, 128), lambda i: (i, 0))
    return pl.pallas_call(
        _head_body,
        grid=grid,
        in_specs=[node_spec, node_spec, node_spec, node_spec,
                  pl.BlockSpec((D, D_OUT), lambda i: (0, 0)),
                  pl.BlockSpec((1, D_OUT), lambda i: (0, 0))],
        out_specs=[
            pl.BlockSpec((blk, D_OUT), lambda i: (i, 0)),
            pl.BlockSpec((blk, D), lambda i: (i, 0)),
            pl.BlockSpec((blk, D), lambda i: (i, 0)),
        ],
        out_shape=[
            jax.ShapeDtypeStruct((N_NODE, D_OUT), _f32),
            jax.ShapeDtypeStruct((N_NODE, D), _f32),
            jax.ShapeDtypeStruct((N_NODE, D), _f32),
        ],
    )(emb_p2[0], emb_p2[1], emb_a2[0], emb_a2[1], w_out, b_out)


_KR = np.kron(np.eye(N_HEADS), np.ones((D_HEAD, 1))).astype(np.float32)


def kernel(x_author, x_paper, edge_index_writes, edge_index_rev,
           W_proj_author, b_proj_author, W_proj_paper, b_proj_paper,
           att_src_writes, att_dst_writes, att_src_rev, att_dst_rev,
           W_k_lin, b_k_lin, q_sem, W_out, b_out):
    kr = jnp.asarray(_KR)
    a_au = jnp.concatenate([att_src_writes.reshape(D, 1) * kr,
                            att_dst_rev.reshape(D, 1) * kr], axis=1)
    a_pa = jnp.concatenate([att_dst_writes.reshape(D, 1) * kr,
                            att_src_rev.reshape(D, 1) * kr], axis=1)

    h_au, au_s = _project(x_author, W_proj_author,
                          b_proj_author.reshape(1, D), a_au)
    h_pa, pa_s = _project(x_paper, W_proj_paper,
                          b_proj_paper.reshape(1, D), a_pa)

    n_extra = E_PAD - N_EDGE
    pad_src = jnp.asarray((np.arange(n_extra) * 131) % N_NODE, jnp.int32)
    pad_dst = jnp.asarray(N_NODE + (np.arange(n_extra) % N_ABS), jnp.int32)

    def _pad_edges(ei):
        ei = ei.astype(jnp.int32)
        return (jnp.concatenate([ei[0], pad_src]),
                jnp.concatenate([ei[1], pad_dst]))

    ei_w = _pad_edges(edge_index_writes)
    ei_r = _pad_edges(edge_index_rev)
    au_s = jnp.pad(au_s, ((0, N_PAD - N_NODE), (0, 0)))
    pa_s = jnp.pad(pa_s, ((0, N_PAD - N_NODE), (0, 0)))
    emb_p2, emb_a2 = _sc_conv(
        h_au.reshape(2 * N_NODE, 128), h_pa.reshape(2 * N_NODE, 128),
        au_s, pa_s, ei_w[0], ei_w[1], ei_r[0], ei_r[1])

    out, emb_p, emb_a = _assemble_head(emb_p2, emb_a2, W_out,
                                       b_out.reshape(1, D_OUT))
    return (out, emb_a, emb_p)
